# Initial kernel scaffold; baseline (speedup 1.0000x reference)
import jax
import jax.numpy as jnp
from jax.experimental import pallas as pl


def kernel(x, edge_index, batch, W1, b1, W2, b2, Wfc, bfc):
    raise NotImplementedError("write your pallas kernel here")


# speedup vs baseline: 11.8451x; 11.8451x over previous
import functools

import jax
import jax.numpy as jnp
from jax import lax
from jax.experimental import pallas as pl
from jax.experimental.pallas import tpu as pltpu
from jax.experimental.pallas import tpu_sc as plsc

N = 10000
E = 320000
D = 128
G = 256

NC = 2
NS = 16
NW = NC * NS
L = 16

EPT = E // NW
CHUNK = 128
NCHUNK = -(-EPT // CHUNK)
EPT_PAD = NCHUNK * CHUNK
ACC_ROWS = 10240
RPT_ACC = ACC_ROWS // NS
WCH = 128

SEG_RPT = 320
SEG_ROWS = NW * SEG_RPT

_SC_MESH = plsc.VectorSubcoreMesh(
    core_axis_name="c", subcore_axis_name="s", num_cores=NC, num_subcores=NS)


@functools.partial(
    pl.kernel,
    out_type=jax.ShapeDtypeStruct((NW, N), jnp.float32),
    mesh=_SC_MESH,
    scratch_types=[
        pltpu.VMEM((EPT,), jnp.int32),
        pltpu.VMEM((N,), jnp.float32),
    ],
    compiler_params=pltpu.CompilerParams(needs_layout_passes=False),
)
def _deg_kernel(dst_hbm, out_hbm, dst_v, hist_v):
    c = lax.axis_index("c")
    s = lax.axis_index("s")
    wid = c * NS + s
    pltpu.sync_copy(dst_hbm.at[wid], dst_v)
    zeros = jnp.zeros((L,), jnp.float32)

    def zloop(i, carry):
        hist_v[pl.ds(i * L, L)] = zeros
        return carry

    lax.fori_loop(0, N // L, zloop, 0)
    ones = jnp.ones((L,), jnp.float32)

    def eloop(i, carry):
        idx = dst_v[pl.ds(i * L, L)]
        plsc.addupdate_scatter(hist_v, [idx], ones)
        return carry

    lax.fori_loop(0, EPT // L, eloop, 0)
    pltpu.sync_copy(hist_v, out_hbm.at[wid])


@functools.partial(
    pl.kernel,
    out_type=jax.ShapeDtypeStruct((NC, ACC_ROWS, D), jnp.float32),
    mesh=_SC_MESH,
    scratch_types=[
        pltpu.VMEM((CHUNK,), jnp.int32),
        pltpu.VMEM((CHUNK,), jnp.int32),
        pltpu.VMEM((CHUNK, D), jnp.float32),
        pltpu.VMEM((WCH, D), jnp.float32),
        pltpu.VMEM_SHARED((ACC_ROWS, D), jnp.float32),
        pltpu.SemaphoreType.DMA,
    ],
    compiler_params=pltpu.CompilerParams(needs_layout_passes=False),
)
def _edge_kernel(y_hbm, srcp_hbm, dstp_hbm, out_hbm,
                 src_v, dst_v, rows_v, buf_v, acc_sh, sem):
    c = lax.axis_index("c")
    s = lax.axis_index("s")
    wid = c * NS + s
    zeros = jnp.zeros((L,), jnp.float32)

    def zloop(i, carry):
        for j in range(D // L):
            buf_v[i, pl.ds(j * L, L)] = zeros
        return carry

    lax.fori_loop(0, WCH, zloop, 0)
    base = s * RPT_ACC
    for k in range(RPT_ACC // WCH):
        pltpu.sync_copy(buf_v, acc_sh.at[pl.ds(base + k * WCH, WCH)])
    plsc.subcore_barrier()

    def chunk(i, carry):
        eb = i * CHUNK
        pltpu.sync_copy(srcp_hbm.at[wid, pl.ds(eb, CHUNK)], src_v)
        pltpu.sync_copy(dstp_hbm.at[wid, pl.ds(eb, CHUNK)], dst_v)
        pltpu.async_copy(y_hbm.at[src_v], rows_v, sem).wait()
        pltpu.sync_copy(rows_v, acc_sh.at[dst_v], add=True)
        return carry

    lax.fori_loop(0, NCHUNK, chunk, 0)
    plsc.subcore_barrier()
    for k in range(RPT_ACC // WCH):
        pltpu.sync_copy(acc_sh.at[pl.ds(base + k * WCH, WCH)], buf_v)
        pltpu.sync_copy(buf_v, out_hbm.at[c, pl.ds(base + k * WCH, WCH)])


@functools.partial(
    pl.kernel,
    out_type=jax.ShapeDtypeStruct((NW, G, D), jnp.float32),
    mesh=_SC_MESH,
    scratch_types=[
        pltpu.VMEM((SEG_RPT, D), jnp.float32),
        pltpu.VMEM((SEG_RPT,), jnp.int32),
        pltpu.VMEM((G, D), jnp.float32),
    ],
    compiler_params=pltpu.CompilerParams(needs_layout_passes=False),
)
def _segmax_kernel(h_hbm, b_hbm, out_hbm, rows_v, bidx_v, out_v):
    c = lax.axis_index("c")
    s = lax.axis_index("s")
    wid = c * NS + s
    pltpu.sync_copy(h_hbm.at[pl.ds(wid * SEG_RPT, SEG_RPT)], rows_v)
    pltpu.sync_copy(b_hbm.at[wid], bidx_v)
    ninf = jnp.full((L,), -jnp.inf, jnp.float32)

    def iloop(i, carry):
        for j in range(D // L):
            out_v[i, pl.ds(j * L, L)] = ninf
        return carry

    lax.fori_loop(0, G, iloop, 0)
    iota = lax.iota(jnp.int32, L)

    def gloop(g, carry):
        seg_prev = carry[0]
        acc = list(carry[1:])
        bv = bidx_v[pl.ds(g * L, L)]
        for rr in range(L):
            r = g * L + rr
            seg = jnp.sum(jnp.where(iota == rr, bv, 0))
            new = seg != seg_prev
            segv = jnp.full((L,), seg, jnp.int32)
            for j in range(D // L):
                rj = plsc.load_gather(rows_v, [jnp.full((L,), r, jnp.int32),
                                               iota + j * L])
                acc[j] = jnp.where(new, rj, jnp.maximum(acc[j], rj))
                plsc.store_scatter(out_v, [segv, iota + j * L], acc[j])
            seg_prev = seg
        return (seg_prev, *acc)

    init = (jnp.int32(-1),) + tuple(ninf for _ in range(D // L))
    lax.fori_loop(0, SEG_RPT // L, gloop, init)
    pltpu.sync_copy(out_v, out_hbm.at[wid])


R1 = 1000


def _mm1_body(degp_ref, x_ref, w_ref, y_ref, dis_ref):
    deg = jnp.sum(degp_ref[...], axis=1) + 1.0
    dis = lax.rsqrt(deg)
    xw = jnp.dot(x_ref[...], w_ref[...], preferred_element_type=jnp.float32,
                 precision=lax.Precision.HIGHEST)
    y_ref[...] = xw * dis[:, None]
    dis_ref[...] = dis[:, None]


_mm1 = pl.pallas_call(
    _mm1_body,
    grid=(N // R1,),
    in_specs=[
        pl.BlockSpec((R1, NW), lambda i: (i, 0)),
        pl.BlockSpec((R1, D), lambda i: (i, 0)),
        pl.BlockSpec((D, D), lambda i: (0, 0)),
    ],
    out_specs=[
        pl.BlockSpec((R1, D), lambda i: (i, 0)),
        pl.BlockSpec((R1, 1), lambda i: (i, 0)),
    ],
    out_shape=[
        jax.ShapeDtypeStruct((N, D), jnp.float32),
        jax.ShapeDtypeStruct((N, 1), jnp.float32),
    ],
)


def _mid_body(accp_ref, y_ref, dis_ref, b_ref, w_ref, y2_ref):
    acc = accp_ref[0] + accp_ref[1] + y_ref[...]
    h = jnp.maximum(acc * dis_ref[...] + b_ref[...], 0.0)
    y2_ref[...] = jnp.dot(h, w_ref[...], preferred_element_type=jnp.float32,
                          precision=lax.Precision.HIGHEST) * dis_ref[...]


_mid = pl.pallas_call(
    _mid_body,
    grid=(N // R1,),
    in_specs=[
        pl.BlockSpec((NC, R1, D), lambda i: (0, i, 0)),
        pl.BlockSpec((R1, D), lambda i: (i, 0)),
        pl.BlockSpec((R1, 1), lambda i: (i, 0)),
        pl.BlockSpec((1, D), lambda i: (0, 0)),
        pl.BlockSpec((D, D), lambda i: (0, 0)),
    ],
    out_specs=pl.BlockSpec((R1, D), lambda i: (i, 0)),
    out_shape=jax.ShapeDtypeStruct((N, D), jnp.float32),
)


def _fin_body(accp_ref, y_ref, dis_ref, b_ref, h_ref):
    acc = accp_ref[0] + accp_ref[1] + y_ref[...]
    h_ref[...] = jnp.maximum(acc * dis_ref[...] + b_ref[...], 0.0)


_fin = pl.pallas_call(
    _fin_body,
    grid=(N // R1,),
    in_specs=[
        pl.BlockSpec((NC, R1, D), lambda i: (0, i, 0)),
        pl.BlockSpec((R1, D), lambda i: (i, 0)),
        pl.BlockSpec((R1, 1), lambda i: (i, 0)),
        pl.BlockSpec((1, D), lambda i: (0, 0)),
    ],
    out_specs=pl.BlockSpec((R1, D), lambda i: (i, 0)),
    out_shape=jax.ShapeDtypeStruct((N, D), jnp.float32),
)


def _pool_body(gp_ref, wfc_ref, bfc_ref, o_ref):
    g = jnp.max(gp_ref[...], axis=0)
    g = jnp.where(g > -jnp.inf, g, 0.0)
    o_ref[...] = jnp.dot(g, wfc_ref[...], preferred_element_type=jnp.float32,
                         precision=lax.Precision.HIGHEST) + bfc_ref[...]


_pool = pl.pallas_call(
    _pool_body,
    out_shape=jax.ShapeDtypeStruct((G, 1), jnp.float32),
)


def kernel(x, edge_index, batch, W1, b1, W2, b2, Wfc, bfc):
    src = edge_index[0]
    dst = edge_index[1]
    dst_r = dst.reshape(NW, EPT)
    degp = _deg_kernel(dst_r)
    y1, dis = _mm1(degp.T, x, W1)

    pad = jnp.zeros((NW, EPT_PAD - EPT), jnp.int32)
    srcp = jnp.concatenate([src.reshape(NW, EPT), pad], axis=1)
    dstp = jnp.concatenate([dst_r, pad + (N + 8)], axis=1)

    acc1 = _edge_kernel(y1, srcp, dstp)
    y2 = _mid(acc1, y1, dis, b1.reshape(1, D), W2)
    acc2 = _edge_kernel(y2, srcp, dstp)
    h2 = _fin(acc2, y2, dis, b2.reshape(1, D))

    h2p = jnp.concatenate(
        [h2, jnp.full((SEG_ROWS - N, D), -jnp.inf, jnp.float32)], axis=0)
    b2d = jnp.concatenate(
        [batch, jnp.full((SEG_ROWS - N,), G - 1, jnp.int32)]).reshape(NW, SEG_RPT)
    gp = _segmax_kernel(h2p, b2d)
    return _pool(gp, Wfc, bfc.reshape(1, 1))
